# Initial kernel scaffold; baseline (speedup 1.0000x reference)
import jax
import jax.numpy as jnp
from jax.experimental import pallas as pl


def kernel(x1, edge_index1, batch1, x2, edge_index2, batch2, cell, W1, a1s, a1d, b1, W2, a2s, a2d, b2, Wg, bg, Wr1, br1, Wr2, br2, Wr3, br3, Wf1, bf1, Wf2, bf2, Wf3, bf3, Wo, bo):
    raise NotImplementedError("write your pallas kernel here")


# speedup vs baseline: 1.0097x; 1.0097x over previous
import functools
import jax
import jax.numpy as jnp
from jax.experimental import pallas as pl

N = 10000
B = 256
H = 10
OD = 64


def _mm(x, w, b=None, act=None):
    M, K = x.shape
    Nc = w.shape[1]
    BM = 1000 if M % 1000 == 0 else M
    if b is None:
        b = jnp.zeros((1, Nc), jnp.float32)
    else:
        b = b.reshape(1, Nc)

    def body(xr, wr, br, outr):
        acc = jnp.dot(xr[...], wr[...], preferred_element_type=jnp.float32)
        acc = acc + br[...]
        if act == "relu":
            acc = jnp.maximum(acc, 0.0)
        elif act == "elu":
            acc = jnp.where(acc > 0, acc, jnp.expm1(acc))
        outr[...] = acc

    return pl.pallas_call(
        body,
        grid=(M // BM,),
        in_specs=[
            pl.BlockSpec((BM, K), lambda m: (m, 0)),
            pl.BlockSpec((K, Nc), lambda m: (0, 0)),
            pl.BlockSpec((1, Nc), lambda m: (0, 0)),
        ],
        out_specs=pl.BlockSpec((BM, Nc), lambda m: (m, 0)),
        out_shape=jax.ShapeDtypeStruct((M, Nc), jnp.float32),
    )(x, w, b)


def _gat(x, ei, W, a_s, a_d, b, heads, oc):
    n = x.shape[0]
    sl = jnp.arange(n, dtype=ei.dtype)
    src = jnp.concatenate([ei[0], sl])
    dst = jnp.concatenate([ei[1], sl])
    h = _mm(x, W)
    hr = h.reshape(n, heads, oc)
    As = jnp.zeros((heads * oc, heads), jnp.float32)
    idx = jnp.arange(heads * oc)
    As = As.at[idx, idx // oc].set(a_s.reshape(-1))
    Ad = jnp.zeros((heads * oc, heads), jnp.float32)
    Ad = Ad.at[idx, idx // oc].set(a_d.reshape(-1))
    a_src = _mm(h, As)
    a_dst = _mm(h, Ad)
    e = jax.nn.leaky_relu(a_src[src] + a_dst[dst], negative_slope=0.2)
    emax = jax.ops.segment_max(e, dst, num_segments=n)
    emax = jnp.where(jnp.isfinite(emax), emax, 0.0)
    ex = jnp.exp(e - emax[dst])
    den = jax.ops.segment_sum(ex, dst, num_segments=n)
    alpha = ex / (den[dst] + 1e-16)
    out = jax.ops.segment_sum(hr[src] * alpha[:, :, None], dst, num_segments=n)
    return out.reshape(n, heads * oc) + b


def _branch(x, ei, batch, W1, a1s, a1d, b1, W2, a2s, a2d, b2, Wg, bg):
    h = jax.nn.elu(_gat(x, ei, W1, a1s, a1d, b1, H, OD))
    h = jax.nn.elu(_gat(h, ei, W2, a2s, a2d, b2, 1, OD))
    g = jax.ops.segment_max(h, batch, num_segments=B)
    g = jnp.where(jnp.isfinite(g), g, 0.0)
    return _mm(g, Wg, bg, act="relu")


def _l2norm(x):
    return x / jnp.clip(jnp.linalg.norm(x, axis=1, keepdims=True), 1e-12, None)


def kernel(x1, edge_index1, batch1, x2, edge_index2, batch2, cell, W1, a1s, a1d, b1, W2, a2s, a2d, b2, Wg, bg, Wr1, br1, Wr2, br2, Wr3, br3, Wf1, bf1, Wf2, bf2, Wf3, bf3, Wo, bo):
    g1 = _branch(x1, edge_index1, batch1, W1, a1s, a1d, b1, W2, a2s, a2d, b2, Wg, bg)
    g2 = _branch(x2, edge_index2, batch2, W1, a1s, a1d, b1, W2, a2s, a2d, b2, Wg, bg)
    c = _l2norm(cell)
    cv = _mm(c, Wr1, br1, act="relu")
    cv = _mm(cv, Wr2, br2, act="relu")
    cv = _mm(cv, Wr3, br3, act="relu")
    xc = jnp.concatenate([g1, g2, cv], axis=1)
    xc = _l2norm(xc)
    xc = _mm(xc, Wf1, bf1, act="relu")
    xc = _mm(xc, Wf2, bf2, act="relu")
    xc = _mm(xc, Wf3, bf3, act="relu")
    out = _mm(xc, Wo, bo)
    return jax.nn.softmax(out, axis=1)
